# baseline (speedup 1.0000x reference)
import functools

import jax
import jax.numpy as jnp
from jax import lax
from jax.experimental import pallas as pl
from jax.experimental.pallas import tpu as pltpu
from jax.experimental.pallas import tpu_sc as plsc

_K = 512
_N = 2 * 576 * 96
_NC = 2
_NS = 16
_NW = _NC * _NS
_PER = _N // _NW
_L = 16
_R = _K * _L


def _prep_body(er_ref, ec_ref, svr_ref, mir_ref):
    a = jnp.broadcast_to(er_ref[...], (_K, _K))
    b = jnp.broadcast_to(ec_ref[...], (_K, _K))
    ii = lax.broadcasted_iota(jnp.int32, (_K, _K), 0)
    kk = lax.broadcasted_iota(jnp.int32, (_K, _K), 1)
    before = (b < a) | ((b == a) & (ii < kk))
    rank = jnp.sum(before.astype(jnp.int32), axis=0, keepdims=True)
    onehot = ii == jnp.broadcast_to(rank, (_K, _K))
    svcol = jnp.sum(jnp.where(onehot, a, 0.0), axis=1, keepdims=True)
    eqv = a == jnp.broadcast_to(svcol, (_K, _K))
    micol = jnp.max(jnp.where(eqv, kk, -1), axis=1, keepdims=True)
    svr_ref[...] = jnp.broadcast_to(svcol, (_K, _L))
    mir_ref[...] = jnp.broadcast_to(micol.astype(jnp.float32), (_K, _L))


_prep = pl.pallas_call(
    _prep_body,
    out_shape=(
        jax.ShapeDtypeStruct((_K, _L), jnp.float32),
        jax.ShapeDtypeStruct((_K, _L), jnp.float32),
    ),
)


def _search_body(h_hbm, svr_hbm, mir_hbm, out_hbm, x_v, o_v, svr_v, mir_v):
    wid = lax.axis_index("s") * _NC + lax.axis_index("c")
    base = wid * _PER
    pltpu.sync_copy(svr_hbm, svr_v)
    pltpu.sync_copy(mir_hbm, mir_v)
    pltpu.sync_copy(h_hbm.at[pl.ds(base, _PER)], x_v)
    lane = lax.iota(jnp.int32, _L)

    @plsc.parallel_loop(0, _PER // _L, unroll=36)
    def body(i):
        x = x_v[pl.ds(i * _L, _L)]
        j16 = jnp.zeros((_L,), jnp.int32)
        step16 = (_K // 2) * _L
        while step16 >= _L:
            v = plsc.load_gather(svr_v, [j16 + (lane + (step16 - _L))])
            j16 = jnp.where(v < x, j16 + step16, j16)
            step16 //= 2
        lovec = jnp.maximum(j16 - _L, 0) + lane
        hivec = j16 + lane
        vlo = plsc.load_gather(svr_v, [lovec])
        vhi = plsc.load_gather(svr_v, [hivec])
        milo = plsc.load_gather(mir_v, [lovec])
        mihi = plsc.load_gather(mir_v, [hivec])
        dlo = jnp.abs(x - vlo)
        dhi = jnp.abs(vhi - x)
        pick_hi = (dhi < dlo) | ((dhi == dlo) & (mihi > milo))
        o_v[pl.ds(i * _L, _L)] = jnp.where(pick_hi, vhi, vlo)

    pltpu.sync_copy(o_v, out_hbm.at[pl.ds(base, _PER)])


@functools.cache
def _make_search():
    mesh = plsc.VectorSubcoreMesh(
        core_axis_name="c", subcore_axis_name="s", num_cores=_NC, num_subcores=_NS
    )
    return pl.kernel(
        _search_body,
        out_type=jax.ShapeDtypeStruct((_N,), jnp.float32),
        mesh=mesh,
        scratch_types=[
            pltpu.VMEM((_PER,), jnp.float32),
            pltpu.VMEM((_PER,), jnp.float32),
            pltpu.VMEM((_R,), jnp.float32),
            pltpu.VMEM((_R,), jnp.float32),
        ],
        compiler_params=pltpu.CompilerParams(needs_layout_passes=False),
    )


def kernel(h, embeddings):
    svr, mir = _prep(embeddings.reshape(1, _K), embeddings.reshape(_K, 1))
    q = _make_search()(h.reshape(_N), svr.reshape(_R), mir.reshape(_R))
    return q.reshape(h.shape)


# speedup vs baseline: 1.0798x; 1.0798x over previous
import functools

import jax
import jax.numpy as jnp
from jax import lax
from jax.experimental import pallas as pl
from jax.experimental.pallas import tpu as pltpu
from jax.experimental.pallas import tpu_sc as plsc

_K = 512
_N = 2 * 576 * 96
_NC = 2
_NS = 16
_NW = _NC * _NS
_PER = _N // _NW
_L = 16
_R = _K * _L


def _prep_body(er_ref, ec_ref, svr_ref, mir_ref):
    a = jnp.broadcast_to(er_ref[...], (_K, _K))
    b = jnp.broadcast_to(ec_ref[...], (_K, _K))
    ii = lax.broadcasted_iota(jnp.int32, (_K, _K), 0)
    kk = lax.broadcasted_iota(jnp.int32, (_K, _K), 1)
    before = (b < a) | ((b == a) & (ii < kk))
    bsum_row = jnp.sum(before.astype(jnp.int32), axis=0, keepdims=True)
    bsum_col = jnp.sum(before.astype(jnp.int32), axis=1, keepdims=True)
    rank_row = bsum_row
    rank_col = (_K - 1) - bsum_col
    eqm = a == b
    macc_col = jnp.max(jnp.where(eqm, kk, -1), axis=1, keepdims=True)
    i64 = lax.broadcasted_iota(jnp.int32, (64, _K), 0)
    r1 = (i64 == jnp.broadcast_to(rank_row // 8, (64, _K))).astype(jnp.float32)
    i8 = lax.broadcasted_iota(jnp.int32, (_K, 8), 1)
    sel = i8 == jnp.broadcast_to(rank_col % 8, (_K, 8))
    r2v = jnp.where(sel, jnp.broadcast_to(ec_ref[...], (_K, 8)), 0.0)
    r2m = jnp.where(sel, jnp.broadcast_to(macc_col.astype(jnp.float32), (_K, 8)), 0.0)
    v = jnp.dot(r1, r2v, preferred_element_type=jnp.float32)
    vm = jnp.dot(r1, r2m, preferred_element_type=jnp.float32)
    g16 = lax.broadcasted_iota(jnp.int32, (8, 128), 1) // _L
    g8 = lax.broadcasted_iota(jnp.int32, (8, 128), 0)
    g = (g16 == g8).astype(jnp.float32)
    svr_ref[...] = jnp.dot(v, g, preferred_element_type=jnp.float32)
    mir_ref[...] = jnp.dot(vm, g, preferred_element_type=jnp.float32)


_prep = pl.pallas_call(
    _prep_body,
    out_shape=(
        jax.ShapeDtypeStruct((64, 128), jnp.float32),
        jax.ShapeDtypeStruct((64, 128), jnp.float32),
    ),
)


def _search_body(h_hbm, svr_hbm, mir_hbm, out_hbm, x_v, o_v, svr_v, mir_v):
    wid = lax.axis_index("s") * _NC + lax.axis_index("c")
    base = wid * _PER
    pltpu.sync_copy(svr_hbm, svr_v)
    pltpu.sync_copy(mir_hbm, mir_v)
    pltpu.sync_copy(h_hbm.at[pl.ds(base, _PER)], x_v)
    lane = lax.iota(jnp.int32, _L)

    @plsc.parallel_loop(0, _PER // _L, unroll=24)
    def body(i):
        x = x_v[pl.ds(i * _L, _L)]
        j16 = jnp.zeros((_L,), jnp.int32)
        step16 = (_K // 2) * _L
        while step16 >= _L:
            v = plsc.load_gather(svr_v, [j16 + (lane + (step16 - _L))])
            j16 = jnp.where(v < x, j16 + step16, j16)
            step16 //= 2
        lovec = jnp.maximum(j16 - _L, 0) + lane
        hivec = j16 + lane
        vlo = plsc.load_gather(svr_v, [lovec])
        vhi = plsc.load_gather(svr_v, [hivec])
        milo = plsc.load_gather(mir_v, [lovec])
        mihi = plsc.load_gather(mir_v, [hivec])
        dlo = jnp.abs(x - vlo)
        dhi = jnp.abs(vhi - x)
        pick_hi = (dhi < dlo) | ((dhi == dlo) & (mihi > milo))
        o_v[pl.ds(i * _L, _L)] = jnp.where(pick_hi, vhi, vlo)

    pltpu.sync_copy(o_v, out_hbm.at[pl.ds(base, _PER)])


@functools.cache
def _make_search():
    mesh = plsc.VectorSubcoreMesh(
        core_axis_name="c", subcore_axis_name="s", num_cores=_NC, num_subcores=_NS
    )
    return pl.kernel(
        _search_body,
        out_type=jax.ShapeDtypeStruct((_N,), jnp.float32),
        mesh=mesh,
        scratch_types=[
            pltpu.VMEM((_PER,), jnp.float32),
            pltpu.VMEM((_PER,), jnp.float32),
            pltpu.VMEM((_R,), jnp.float32),
            pltpu.VMEM((_R,), jnp.float32),
        ],
        compiler_params=pltpu.CompilerParams(needs_layout_passes=False),
    )


def kernel(h, embeddings):
    svr, mir = _prep(embeddings.reshape(1, _K), embeddings.reshape(_K, 1))
    q = _make_search()(h.reshape(_N), svr.reshape(_R), mir.reshape(_R))
    return q.reshape(h.shape)
